# baseline (speedup 1.0000x reference)
import functools

import jax
import jax.numpy as jnp
from jax import lax
from jax.experimental import pallas as pl
from jax.experimental.pallas import tpu as pltpu
from jax.experimental.pallas import tpu_sc as plsc

NUM_WORKERS = 32
CHUNK = 832
L = 16


def _transpose_block(in_buf, out_buf, ncols):
    iota = lax.iota(jnp.int32, L)
    for u in range(ncols // 2):
        for h in range(2):
            c = jnp.full((L,), 2 * u + h, jnp.int32)
            for k in range(4):
                vals = plsc.load_gather(in_buf, [iota + L * k, c])
                out_buf[u, pl.ds(64 * h + L * k, L)] = vals


@jax.jit
def _sc_linearize(table_t, tail_t):
    f, v = table_t.shape
    nfull = v // 128
    rem = v - nfull * 128

    mesh = plsc.VectorSubcoreMesh(core_axis_name="c", subcore_axis_name="s")

    @functools.partial(
        pl.kernel,
        mesh=mesh,
        out_type=jax.ShapeDtypeStruct((v // 2, 128), jnp.float32),
        scratch_types=[
            pltpu.VMEM((64, 128), jnp.float32),
            pltpu.VMEM((64, 128), jnp.float32),
        ],
        compiler_params=pltpu.CompilerParams(needs_layout_passes=False),
    )
    def linearize(t_hbm, tail_hbm, out_hbm, in_buf, out_buf):
        wid = lax.axis_index("s") * 2 + lax.axis_index("c")
        nb = (nfull - wid + NUM_WORKERS - 1) // NUM_WORKERS

        @pl.loop(0, nb)
        def _(i):
            b = wid + NUM_WORKERS * i
            pltpu.sync_copy(t_hbm.at[:, pl.ds(b * 128, 128)], in_buf)
            _transpose_block(in_buf, out_buf, 128)
            pltpu.sync_copy(out_buf, out_hbm.at[pl.ds(b * 64, 64), :])

        if rem:
            @pl.when(wid == NUM_WORKERS - 1)
            def _():
                pltpu.sync_copy(tail_hbm, in_buf)
                _transpose_block(in_buf, out_buf, 2 * (rem // 2))
                pltpu.sync_copy(
                    out_buf.at[pl.ds(0, rem // 2), :],
                    out_hbm.at[pl.ds(nfull * 64, rem // 2), :])

    return linearize(table_t, tail_t)


@functools.partial(jax.jit, static_argnames=("n", "f"))
def _sc_gather(table, idx, *, n, f):
    b_per_w = n // NUM_WORKERS
    n_chunks = b_per_w // CHUNK

    mesh = plsc.VectorSubcoreMesh(core_axis_name="c", subcore_axis_name="s")

    @functools.partial(
        pl.kernel,
        mesh=mesh,
        out_type=jax.ShapeDtypeStruct((n, f), jnp.float32),
        scratch_types=[
            pltpu.VMEM((b_per_w,), jnp.int32),
            pltpu.VMEM((CHUNK, f), jnp.float32),
            pltpu.VMEM((CHUNK, f), jnp.float32),
            pltpu.SemaphoreType.DMA,
            pltpu.SemaphoreType.DMA,
            pltpu.SemaphoreType.DMA,
            pltpu.SemaphoreType.DMA,
        ],
        compiler_params=pltpu.CompilerParams(use_tc_tiling_on_sc=False),
    )
    def gather(table_hbm, idx_hbm, out_hbm, idx_v, rows0, rows1,
               gsem0, gsem1, ssem0, ssem1):
        wid = lax.axis_index("s") * 2 + lax.axis_index("c")
        base = wid * b_per_w
        rows = (rows0, rows1)
        gsem = (gsem0, gsem1)
        ssem = (ssem0, ssem1)

        pltpu.sync_copy(idx_hbm.at[pl.ds(base, b_per_w)], idx_v)

        def start_gather(j):
            return pltpu.async_copy(
                table_hbm.at[idx_v.at[pl.ds(j * CHUNK, CHUNK)]],
                rows[j % 2], gsem[j % 2])

        def start_store(j):
            return pltpu.async_copy(
                rows[j % 2], out_hbm.at[pl.ds(base + j * CHUNK, CHUNK)],
                ssem[j % 2])

        g = [None] * n_chunks
        s = [None] * n_chunks
        g[0] = start_gather(0)
        for j in range(n_chunks):
            g[j].wait()
            s[j] = start_store(j)
            if j + 1 < n_chunks:
                if j >= 1:
                    s[j - 1].wait()
                g[j + 1] = start_gather(j + 1)
        s[n_chunks - 2].wait()
        s[n_chunks - 1].wait()

    return gather(table, idx)


def kernel(image, W):
    B, A = image.shape
    V, F = W.shape
    n = B * A
    idx = image.reshape(n).astype(jnp.int32)
    Wt = W.T
    nfull = V // 128
    rem = V - nfull * 128
    tail = lax.slice(Wt, (0, nfull * 128), (F, V))
    tail = jnp.pad(tail, ((0, 0), (0, 128 - rem)))
    table = _sc_linearize(Wt, tail).reshape(V, F)
    out = _sc_gather(table, idx, n=n, f=F)
    return out.reshape(B, A * F)


# speedup vs baseline: 1.4474x; 1.4474x over previous
import functools

import jax
import jax.numpy as jnp
from jax import lax
from jax.experimental import pallas as pl
from jax.experimental.pallas import tpu as pltpu
from jax.experimental.pallas import tpu_sc as plsc

NUM_WORKERS = 32
CHUNK = 832
L = 16


def _transpose_block(in_buf, out_buf, iotas, ncols):
    for u in range(ncols // 2):
        vals = []
        for h in range(2):
            c = jnp.full((L,), 2 * u + h, jnp.int32)
            vals += [plsc.load_gather(in_buf, [iotas[k], c])
                     for k in range(4)]
        for h in range(2):
            for k in range(4):
                out_buf[u, pl.ds(64 * h + L * k, L)] = vals[4 * h + k]


@jax.jit
def _sc_linearize(table_t, tail_t):
    f, v = table_t.shape
    nfull = v // 128
    rem = v - nfull * 128
    per_w = nfull // NUM_WORKERS
    extra = nfull - per_w * NUM_WORKERS

    mesh = plsc.VectorSubcoreMesh(core_axis_name="c", subcore_axis_name="s")

    @functools.partial(
        pl.kernel,
        mesh=mesh,
        out_type=jax.ShapeDtypeStruct((v // 2, 128), jnp.float32),
        scratch_types=[
            pltpu.VMEM((64, 128), jnp.float32),
            pltpu.VMEM((64, 128), jnp.float32),
            pltpu.VMEM((64, 128), jnp.float32),
            pltpu.VMEM((64, 128), jnp.float32),
            pltpu.SemaphoreType.DMA,
            pltpu.SemaphoreType.DMA,
        ],
        compiler_params=pltpu.CompilerParams(needs_layout_passes=False),
    )
    def linearize(t_hbm, tail_hbm, out_hbm, in0, in1, out0, out1,
                  g0, g1):
        wid = lax.axis_index("s") * 2 + lax.axis_index("c")
        lo = wid * per_w + jnp.minimum(wid, extra)
        cnt = per_w + jnp.where(wid < extra, 1, 0)
        hi = lo + cnt
        iotas = [lax.iota(jnp.int32, L) + L * k for k in range(4)]

        def start_in(b, buf, sem):
            return pltpu.async_copy(
                t_hbm.at[:, pl.ds(b * 128, 128)], buf, sem)

        start_in(lo, in0, g0)

        @pl.when(cnt > 1)
        def _():
            start_in(lo + 1, in1, g1)

        def wait_in(buf, sem):
            pltpu.make_async_copy(
                t_hbm.at[:, pl.ds(0, 128)], buf, sem).wait()

        npairs = (cnt + 1) // 2

        @pl.loop(0, npairs)
        def _(jj):
            b = lo + 2 * jj

            wait_in(in0, g0)
            _transpose_block(in0, out0, iotas, 128)

            @pl.when(b + 2 < hi)
            def _():
                start_in(b + 2, in0, g0)

            pltpu.sync_copy(out0, out_hbm.at[pl.ds(b * 64, 64), :])

            @pl.when(b + 1 < hi)
            def _():
                wait_in(in1, g1)
                _transpose_block(in1, out1, iotas, 128)

                @pl.when(b + 3 < hi)
                def _():
                    start_in(b + 3, in1, g1)

                pltpu.sync_copy(out1, out_hbm.at[pl.ds((b + 1) * 64, 64), :])

        if rem:
            @pl.when(wid == NUM_WORKERS - 1)
            def _():
                pltpu.sync_copy(tail_hbm, in0)
                _transpose_block(in0, out0, iotas, 2 * (rem // 2))
                pltpu.sync_copy(
                    out0.at[pl.ds(0, rem // 2), :],
                    out_hbm.at[pl.ds(nfull * 64, rem // 2), :])

    return linearize(table_t, tail_t)


@functools.partial(jax.jit, static_argnames=("n", "f"))
def _sc_gather(table, idx, *, n, f):
    b_per_w = n // NUM_WORKERS
    n_chunks = b_per_w // CHUNK

    mesh = plsc.VectorSubcoreMesh(core_axis_name="c", subcore_axis_name="s")

    @functools.partial(
        pl.kernel,
        mesh=mesh,
        out_type=jax.ShapeDtypeStruct((n, f), jnp.float32),
        scratch_types=[
            pltpu.VMEM((b_per_w,), jnp.int32),
            pltpu.VMEM((CHUNK, f), jnp.float32),
            pltpu.VMEM((CHUNK, f), jnp.float32),
            pltpu.SemaphoreType.DMA,
            pltpu.SemaphoreType.DMA,
            pltpu.SemaphoreType.DMA,
            pltpu.SemaphoreType.DMA,
        ],
        compiler_params=pltpu.CompilerParams(use_tc_tiling_on_sc=False),
    )
    def gather(table_hbm, idx_hbm, out_hbm, idx_v, rows0, rows1,
               gsem0, gsem1, ssem0, ssem1):
        wid = lax.axis_index("s") * 2 + lax.axis_index("c")
        base = wid * b_per_w
        rows = (rows0, rows1)
        gsem = (gsem0, gsem1)
        ssem = (ssem0, ssem1)

        pltpu.sync_copy(idx_hbm.at[pl.ds(base, b_per_w)], idx_v)

        def start_gather(j):
            return pltpu.async_copy(
                table_hbm.at[idx_v.at[pl.ds(j * CHUNK, CHUNK)]],
                rows[j % 2], gsem[j % 2])

        def start_store(j):
            return pltpu.async_copy(
                rows[j % 2], out_hbm.at[pl.ds(base + j * CHUNK, CHUNK)],
                ssem[j % 2])

        g = [None] * n_chunks
        s = [None] * n_chunks
        g[0] = start_gather(0)
        for j in range(n_chunks):
            g[j].wait()
            s[j] = start_store(j)
            if j + 1 < n_chunks:
                if j >= 1:
                    s[j - 1].wait()
                g[j + 1] = start_gather(j + 1)
        s[n_chunks - 2].wait()
        s[n_chunks - 1].wait()

    return gather(table, idx)


def kernel(image, W):
    B, A = image.shape
    V, F = W.shape
    n = B * A
    idx = image.reshape(n).astype(jnp.int32)
    Wt = W.T
    nfull = V // 128
    rem = V - nfull * 128
    tail = lax.slice(Wt, (0, nfull * 128), (F, V))
    tail = jnp.pad(tail, ((0, 0), (0, 128 - rem)))
    table = _sc_linearize(Wt, tail).reshape(V, F)
    out = _sc_gather(table, idx, n=n, f=F)
    return out.reshape(B, A * F)
